# initial kernel scaffold (unmeasured)
import jax
import jax.numpy as jnp
from jax import lax
from jax.experimental import pallas as pl
from jax.experimental.pallas import tpu as pltpu

CHUNK = 256
MAX_CHUNKS = 16


def kernel(x, dest):
    n, d = x.shape
    my_x = lax.axis_index("x")

    send_mask = dest != my_x
    n_send = jnp.sum(send_mask.astype(jnp.int32))
    n_keep = n - n_send
    n_recv = n_send

    xb = x.astype(jnp.bfloat16)
    perm_send = jnp.argsort(jnp.where(send_mask, 0, 1))
    perm_keep = jnp.argsort(jnp.where(send_mask, 1, 0))
    send_buf = jnp.take(xb, perm_send, axis=0)
    keep_buf = jnp.take(xb, perm_keep, axis=0)

    n_chunks = (n_send + CHUNK - 1) // CHUNK
    scalars = jnp.reshape(n_chunks.astype(jnp.int32), (1,))

    def body(nc_ref, send_ref, recv_ref, send_sems, recv_sems):
        mx = lax.axis_index("x")
        my = lax.axis_index("y")
        mz = lax.axis_index("z")
        partner = (1 - mx, my, mz)

        barrier_sem = pltpu.get_barrier_semaphore()
        pl.semaphore_signal(
            barrier_sem, inc=1,
            device_id=partner, device_id_type=pl.DeviceIdType.MESH,
        )
        pl.semaphore_wait(barrier_sem, 1)

        nc = nc_ref[0]

        def chunk_rdma(i):
            return pltpu.make_async_remote_copy(
                src_ref=send_ref.at[pl.ds(i * CHUNK, CHUNK)],
                dst_ref=recv_ref.at[pl.ds(i * CHUNK, CHUNK)],
                send_sem=send_sems.at[i],
                recv_sem=recv_sems.at[i],
                device_id=partner,
                device_id_type=pl.DeviceIdType.MESH,
            )

        for i in range(MAX_CHUNKS):
            @pl.when(i < nc)
            def _():
                chunk_rdma(i).start()

        for i in range(MAX_CHUNKS):
            @pl.when(i < nc)
            def _():
                chunk_rdma(i).wait()

    recv = pl.pallas_call(
        body,
        out_shape=jax.ShapeDtypeStruct((n, d), jnp.bfloat16),
        in_specs=[
            pl.BlockSpec(memory_space=pltpu.SMEM),
            pl.BlockSpec(memory_space=pltpu.VMEM),
        ],
        out_specs=pl.BlockSpec(memory_space=pltpu.VMEM),
        scratch_shapes=[
            pltpu.SemaphoreType.DMA((MAX_CHUNKS,)),
            pltpu.SemaphoreType.DMA((MAX_CHUNKS,)),
        ],
        compiler_params=pltpu.CompilerParams(collective_id=0),
    )(scalars, send_buf)

    ar = jnp.arange(n, dtype=jnp.int32)

    def assemble(first, second, n_first):
        idx2 = jnp.clip(ar - n_first, 0, n - 1)
        return jnp.where(
            (ar < n_first)[:, None],
            first,
            jnp.take(second, idx2, axis=0),
        )

    return lax.cond(
        my_x == 0,
        lambda: assemble(keep_buf, recv, n_keep),
        lambda: assemble(recv, keep_buf, n_recv),
    )


# baseline (device time: 81764 ns/iter reference)
import jax
import jax.numpy as jnp
from jax import lax
from jax.experimental import pallas as pl
from jax.experimental.pallas import tpu as pltpu

G = 8
N_BITS = 13
BLK = 512


def kernel(x, dest):
    n, d = x.shape
    P = n + 2 * G
    my_x = lax.axis_index("x")

    is0 = (dest == 0).astype(jnp.int32)
    c0 = jnp.sum(is0)
    A = (c0 + G - 1) // G * G
    pos = jnp.where(is0, jnp.cumsum(is0) - 1, A + jnp.cumsum(1 - is0) - 1)
    packed = jnp.zeros((P, d), jnp.bfloat16).at[pos].set(x.astype(jnp.bfloat16))

    L = jnp.where(my_x == 0, n - c0, c0)
    src_base = jnp.where(my_x == 0, A, 0)
    nL = n - L
    dst_base = jnp.where(my_x == 0, 0, (nL + G - 1) // G * G)
    L_up = (L + G - 1) // G * G
    scalars = jnp.stack([L_up, src_base, dst_base, c0, A - c0]).astype(jnp.int32)

    def body(s_ref, packed_ref, out_ref, comm_ref, copy_sem, send_sems, recv_sems):
        mx = lax.axis_index("x")
        my = lax.axis_index("y")
        mz = lax.axis_index("z")
        partner = (1 - mx, my, mz)

        L_ = s_ref[0]
        src_base_ = s_ref[1]
        dst_base_ = s_ref[2]
        c0_ = s_ref[3]
        gap_ = s_ref[4]

        cp = pltpu.make_async_copy(packed_ref, comm_ref, copy_sem)
        cp.start()
        cp.wait()

        barrier_sem = pltpu.get_barrier_semaphore()
        pl.semaphore_signal(
            barrier_sem, inc=1,
            device_id=partner, device_id_type=pl.DeviceIdType.MESH,
        )
        pl.semaphore_wait(barrier_sem, 1)

        def block_rdma(sem_idx, off, sz):
            src_off = pl.multiple_of(src_base_ + off, G)
            dst_off = pl.multiple_of(dst_base_ + off, G)
            return pltpu.make_async_remote_copy(
                src_ref=packed_ref.at[pl.ds(src_off, sz)],
                dst_ref=comm_ref.at[pl.ds(dst_off, sz)],
                send_sem=send_sems.at[sem_idx],
                recv_sem=recv_sems.at[sem_idx],
                device_id=partner,
                device_id_type=pl.DeviceIdType.MESH,
            )

        def schedule(action):
            off = jnp.int32(0)
            for b in range(N_BITS - 1, 2, -1):
                bit = (L_ >> b) & 1

                @pl.when(bit == 1)
                def _():
                    action(block_rdma(b, off, 1 << b))

                off = off + bit * (1 << b)

        schedule(lambda r: r.start())
        schedule(lambda r: r.wait())

        for g in range(G):
            @pl.when(gap_ == g)
            def _():
                for t in range(n // BLK):
                    blk = comm_ref[pl.ds(t * BLK, BLK + G), :]
                    rows = t * BLK + lax.broadcasted_iota(
                        jnp.int32, (BLK, 1), 0
                    )
                    out_ref[pl.ds(t * BLK, BLK), :] = jnp.where(
                        rows < c0_, blk[:BLK], blk[g:g + BLK]
                    )

    return pl.pallas_call(
        body,
        out_shape=jax.ShapeDtypeStruct((n, d), jnp.bfloat16),
        in_specs=[
            pl.BlockSpec(memory_space=pltpu.SMEM),
            pl.BlockSpec(memory_space=pltpu.VMEM),
        ],
        out_specs=pl.BlockSpec(memory_space=pltpu.VMEM),
        scratch_shapes=[
            pltpu.VMEM((P, d), jnp.bfloat16),
            pltpu.SemaphoreType.DMA,
            pltpu.SemaphoreType.DMA((N_BITS,)),
            pltpu.SemaphoreType.DMA((N_BITS,)),
        ],
        compiler_params=pltpu.CompilerParams(collective_id=0),
    )(scalars, packed)


# device time: 81255 ns/iter; 1.0063x vs baseline; 1.0063x over previous
import jax
import jax.numpy as jnp
from jax import lax
from jax.experimental import pallas as pl
from jax.experimental.pallas import tpu as pltpu

G = 8
N_BITS = 13
BLK = 512


def kernel(x, dest):
    n, d = x.shape
    P = n + 2 * G
    my_x = lax.axis_index("x")

    is0 = (dest == 0).astype(jnp.int32)
    c0 = jnp.sum(is0)
    A = (c0 + G - 1) // G * G
    pos = jnp.where(is0, jnp.cumsum(is0) - 1, A + jnp.cumsum(1 - is0) - 1)
    packed = jnp.zeros((P, d), jnp.bfloat16).at[pos].set(x.astype(jnp.bfloat16))

    L = jnp.where(my_x == 0, n - c0, c0)
    src_base = jnp.where(my_x == 0, A, 0)
    nL = n - L
    dst_base = jnp.where(my_x == 0, 0, (nL + G - 1) // G * G)
    L_up = (L + G - 1) // G * G
    keep_cnt = jnp.where(my_x == 0, c0, n - c0)
    keep_base = jnp.where(my_x == 0, 0, A)
    keep_len = (keep_cnt + G - 1) // G * G
    scalars = jnp.stack(
        [L_up, src_base, dst_base, c0, A - c0, keep_base, keep_len]
    ).astype(jnp.int32)

    def body(s_ref, packed_ref, out_ref, comm_ref, copy_sems, send_sems, recv_sems):
        mx = lax.axis_index("x")
        my = lax.axis_index("y")
        mz = lax.axis_index("z")
        partner = (1 - mx, my, mz)

        L_ = s_ref[0]
        src_base_ = s_ref[1]
        dst_base_ = s_ref[2]
        c0_ = s_ref[3]
        gap_ = s_ref[4]
        keep_base_ = s_ref[5]
        keep_len_ = s_ref[6]

        barrier_sem = pltpu.get_barrier_semaphore()
        pl.semaphore_signal(
            barrier_sem, inc=1,
            device_id=partner, device_id_type=pl.DeviceIdType.MESH,
        )
        pl.semaphore_wait(barrier_sem, 1)

        def block_rdma(sem_idx, off, sz):
            src_off = pl.multiple_of(src_base_ + off, G)
            dst_off = pl.multiple_of(dst_base_ + off, G)
            return pltpu.make_async_remote_copy(
                src_ref=packed_ref.at[pl.ds(src_off, sz)],
                dst_ref=comm_ref.at[pl.ds(dst_off, sz)],
                send_sem=send_sems.at[sem_idx],
                recv_sem=recv_sems.at[sem_idx],
                device_id=partner,
                device_id_type=pl.DeviceIdType.MESH,
            )

        def keep_copy(b, off):
            o = pl.multiple_of(keep_base_ + off, G)
            sz = 1 << b
            return pltpu.make_async_copy(
                packed_ref.at[pl.ds(o, sz)],
                comm_ref.at[pl.ds(o, sz)],
                copy_sems.at[b],
            )

        def schedule(length, action):
            off = jnp.int32(0)
            for b in range(N_BITS - 1, 2, -1):
                bit = (length >> b) & 1

                @pl.when(bit == 1)
                def _():
                    action(b, off)

                off = off + bit * (1 << b)

        schedule(L_, lambda b, off: block_rdma(b, off, 1 << b).start())
        schedule(keep_len_, lambda b, off: keep_copy(b, off).start())
        schedule(keep_len_, lambda b, off: keep_copy(b, off).wait())
        schedule(L_, lambda b, off: block_rdma(b, off, 1 << b).wait())

        for g in range(G):
            @pl.when(gap_ == g)
            def _():
                for t in range(n // BLK):
                    blk = comm_ref[pl.ds(t * BLK, BLK + G), :]
                    rows = t * BLK + lax.broadcasted_iota(
                        jnp.int32, (BLK, 1), 0
                    )
                    out_ref[pl.ds(t * BLK, BLK), :] = jnp.where(
                        rows < c0_, blk[:BLK], blk[g:g + BLK]
                    )

    return pl.pallas_call(
        body,
        out_shape=jax.ShapeDtypeStruct((n, d), jnp.bfloat16),
        in_specs=[
            pl.BlockSpec(memory_space=pltpu.SMEM),
            pl.BlockSpec(memory_space=pltpu.VMEM),
        ],
        out_specs=pl.BlockSpec(memory_space=pltpu.VMEM),
        scratch_shapes=[
            pltpu.VMEM((P, d), jnp.bfloat16),
            pltpu.SemaphoreType.DMA((N_BITS,)),
            pltpu.SemaphoreType.DMA((N_BITS,)),
            pltpu.SemaphoreType.DMA((N_BITS,)),
        ],
        compiler_params=pltpu.CompilerParams(collective_id=0),
    )(scalars, packed)
